# baseline (speedup 1.0000x reference)
import functools

import jax
import jax.numpy as jnp
from jax import lax
from jax.experimental import pallas as pl
from jax.experimental.pallas import tpu as pltpu
from jax.experimental.pallas import tpu_sc as plsc

N_NODES = 10000
DIM = 128

NUM_TILES = 32
CHUNK = 80
NPAD = 10240
ROWS_PER_TILE = NPAD // 16


def _mm_body(x_ref, w_ref, asrc_ref, adst_ref, h_ref, as_ref, ad_ref,
             ms_ref, md_ref):
  h = jax.lax.dot_general(x_ref[...], w_ref[...], (((1,), (0,)), ((), ())),
                          preferred_element_type=jnp.float32)
  h_ref[...] = h
  a_s = jnp.sum(h * asrc_ref[...], axis=1, keepdims=True)
  a_d = jnp.sum(h * adst_ref[...], axis=1, keepdims=True)
  as_ref[...] = a_s
  ad_ref[...] = a_d

  @pl.when(pl.program_id(0) == 0)
  def _():
    ms_ref[0, 0] = jnp.float32(-jnp.inf)
    md_ref[0, 0] = jnp.float32(-jnp.inf)

  ms_ref[0, 0] = jnp.maximum(ms_ref[0, 0], jnp.max(a_s))
  md_ref[0, 0] = jnp.maximum(md_ref[0, 0], jnp.max(a_d))


def _project(x, W, att_src, att_dst):
  blk = 1000
  grid = N_NODES // blk
  return pl.pallas_call(
      _mm_body,
      grid=(grid,),
      in_specs=[
          pl.BlockSpec((blk, DIM), lambda i: (i, 0)),
          pl.BlockSpec((DIM, DIM), lambda i: (0, 0)),
          pl.BlockSpec((1, DIM), lambda i: (0, 0)),
          pl.BlockSpec((1, DIM), lambda i: (0, 0)),
      ],
      out_specs=[
          pl.BlockSpec((blk, DIM), lambda i: (i, 0)),
          pl.BlockSpec((blk, 1), lambda i: (i, 0)),
          pl.BlockSpec((blk, 1), lambda i: (i, 0)),
          pl.BlockSpec((1, 1), lambda i: (0, 0), memory_space=pltpu.SMEM),
          pl.BlockSpec((1, 1), lambda i: (0, 0), memory_space=pltpu.SMEM),
      ],
      out_shape=[
          jax.ShapeDtypeStruct((NPAD, DIM), jnp.float32),
          jax.ShapeDtypeStruct((NPAD, 1), jnp.float32),
          jax.ShapeDtypeStruct((NPAD, 1), jnp.float32),
          jax.ShapeDtypeStruct((1, 1), jnp.float32),
          jax.ShapeDtypeStruct((1, 1), jnp.float32),
      ],
  )(x, W, att_src.reshape(1, DIM), att_dst.reshape(1, DIM))


def _sc_w_body(src_hbm, dst_hbm, asrc_hbm, adst_hbm, m_hbm, z1d_hbm,
               w_hbm, den_hbm,
               src_v, dst_v, w_v, asrc_v, adst_v, den_v, m_v):
  core = lax.axis_index("c")
  sub = lax.axis_index("s")
  wid = sub * 2 + core

  n_groups = src_hbm.shape[1]
  grp_chunks = src_hbm.shape[2]

  pltpu.sync_copy(asrc_hbm, asrc_v)
  pltpu.sync_copy(adst_hbm, adst_v)
  pltpu.sync_copy(z1d_hbm, den_v)
  pltpu.sync_copy(m_hbm, m_v)
  mv = m_v[...]

  def group_step(grp, carry):
    pltpu.sync_copy(src_hbm.at[wid, grp], src_v)
    pltpu.sync_copy(dst_hbm.at[wid, grp], dst_v)

    def chunk_step(j, carry2):
      for g in range(CHUNK // 16):
        s16 = src_v[j, pl.ds(g * 16, 16)]
        d16 = dst_v[j, pl.ds(g * 16, 16)]
        av = plsc.load_gather(asrc_v, [s16])
        bv = plsc.load_gather(adst_v, [d16])
        s = av + bv
        s = jnp.maximum(s, 0.2 * s)
        wv = jnp.exp(s - mv)
        plsc.addupdate_scatter(den_v, [d16], wv)
        w_v[j, pl.ds(g * 16, 16)] = wv
      return carry2

    lax.fori_loop(0, grp_chunks, chunk_step, 0)
    pltpu.sync_copy(w_v, w_hbm.at[wid, grp])
    return carry

  lax.fori_loop(0, n_groups, group_step, 0)
  pltpu.sync_copy(den_v, den_hbm.at[wid])


def _sc_weights(src4, dst4, a_src, a_dst, m_arr, z1d):
  grp_chunks = src4.shape[2]
  mesh = plsc.VectorSubcoreMesh(core_axis_name="c", subcore_axis_name="s")
  kern = functools.partial(
      pl.kernel,
      mesh=mesh,
      compiler_params=pltpu.CompilerParams(needs_layout_passes=False),
      out_type=[
          jax.ShapeDtypeStruct(src4.shape, jnp.float32),
          jax.ShapeDtypeStruct((NUM_TILES, NPAD), jnp.float32),
      ],
      scratch_types=[
          pltpu.VMEM((grp_chunks, CHUNK), jnp.int32),
          pltpu.VMEM((grp_chunks, CHUNK), jnp.int32),
          pltpu.VMEM((grp_chunks, CHUNK), jnp.float32),
          pltpu.VMEM((NPAD,), jnp.float32),
          pltpu.VMEM((NPAD,), jnp.float32),
          pltpu.VMEM((NPAD,), jnp.float32),
          pltpu.VMEM((16,), jnp.float32),
      ],
  )(_sc_w_body)
  return kern(src4, dst4, a_src, a_dst, m_arr, z1d)


def _sc_agg_body(src_hbm, dst_hbm, w_hbm, h_hbm, zrows_hbm,
                 num_hbm,
                 src_v, dst_v, w_v, rows0, rows1, rows2,
                 num_sh, sg0, sg1, sg2, ss0, ss1, ss2):
  core = lax.axis_index("c")
  sub = lax.axis_index("s")
  wid = sub * 2 + core

  n_groups = src_hbm.shape[1]
  grp_chunks = src_hbm.shape[2]
  rows = [rows0, rows1, rows2]
  sg = [sg0, sg1, sg2]
  ss = [ss0, ss1, ss2]

  row0 = sub * ROWS_PER_TILE
  for i in range(5):
    pltpu.sync_copy(zrows_hbm, num_sh.at[pl.ds(row0 + i * 128, 128)])
  plsc.subcore_barrier()

  def group_step(grp, carry):
    pltpu.sync_copy(src_hbm.at[wid, grp], src_v)
    pltpu.sync_copy(dst_hbm.at[wid, grp], dst_v)
    pltpu.sync_copy(w_hbm.at[wid, grp], w_v)

    cp_g = {}
    cp_s = {}
    for j in range(grp_chunks):
      b = j % 3
      if j == 0:
        cp_g[0] = pltpu.async_copy(h_hbm.at[src_v.at[0]], rows[0], sg[0])
        cp_g[1] = pltpu.async_copy(h_hbm.at[src_v.at[1]], rows[1], sg[1])
      cp_g[j].wait()
      if j + 2 < grp_chunks:
        b2 = (j + 2) % 3
        if j >= 1:
          cp_s[j - 1].wait()
        cp_g[j + 2] = pltpu.async_copy(
            h_hbm.at[src_v.at[j + 2]], rows[b2], sg[b2])

      jv = jnp.full((16,), j, jnp.int32)

      def scale_row(r, c, b=b, jv=jv):
        wb = plsc.load_gather(w_v, [jv, jnp.full((16,), r, jnp.int32)])
        for d in range(DIM // 16):
          rows[b][r, pl.ds(d * 16, 16)] = rows[b][r, pl.ds(d * 16, 16)] * wb
        return c

      lax.fori_loop(0, CHUNK, scale_row, 0, unroll=8)

      cp_s[j] = pltpu.async_copy(rows[b], num_sh.at[dst_v.at[j]], ss[b],
                                 add=True)

    for j in range(grp_chunks - 3, grp_chunks):
      cp_s[j].wait()
    return carry

  lax.fori_loop(0, n_groups, group_step, 0)

  plsc.subcore_barrier()
  pltpu.sync_copy(num_sh.at[pl.ds(row0, ROWS_PER_TILE)],
                  num_hbm.at[core, pl.ds(row0, ROWS_PER_TILE)])


def _sc_aggregate(src4, dst4, w4, h, zrows):
  grp_chunks = src4.shape[2]
  mesh = plsc.VectorSubcoreMesh(core_axis_name="c", subcore_axis_name="s")
  kern = functools.partial(
      pl.kernel,
      mesh=mesh,
      compiler_params=pltpu.CompilerParams(needs_layout_passes=False),
      out_type=jax.ShapeDtypeStruct((2, NPAD, DIM), jnp.float32),
      scratch_types=[
          pltpu.VMEM((grp_chunks, CHUNK), jnp.int32),
          pltpu.VMEM((grp_chunks, CHUNK), jnp.int32),
          pltpu.VMEM((grp_chunks, CHUNK), jnp.float32),
          pltpu.VMEM((CHUNK, DIM), jnp.float32),
          pltpu.VMEM((CHUNK, DIM), jnp.float32),
          pltpu.VMEM((CHUNK, DIM), jnp.float32),
          pltpu.VMEM_SHARED((NPAD, DIM), jnp.float32),
          pltpu.SemaphoreType.DMA,
          pltpu.SemaphoreType.DMA,
          pltpu.SemaphoreType.DMA,
          pltpu.SemaphoreType.DMA,
          pltpu.SemaphoreType.DMA,
          pltpu.SemaphoreType.DMA,
      ],
  )(_sc_agg_body)
  return kern(src4, dst4, w4, h, zrows)


def _fin_body(num_ref, den_ref, h_ref, as_ref, ad_ref, bias_ref, m_ref,
              out_ref):
  s = as_ref[...] + ad_ref[...]
  s = jnp.maximum(s, 0.2 * s)
  ws = jnp.exp(s - m_ref[0, 0])
  num = num_ref[0] + num_ref[1] + ws * h_ref[...]
  blk = h_ref.shape[0]
  den_blk = den_ref[:, pl.ds(pl.program_id(0) * blk, blk)]
  den = jnp.sum(den_blk, axis=0)[:, None] + ws + 1e-16
  out_ref[...] = num / den + bias_ref[...]


def _finalize(num_part, den_part, h, a_s, a_d, bias, m_scalar):
  blk = 1024
  grid = NPAD // blk
  return pl.pallas_call(
      _fin_body,
      grid=(grid,),
      in_specs=[
          pl.BlockSpec((2, blk, DIM), lambda i: (0, i, 0)),
          pl.BlockSpec((NUM_TILES, NPAD), lambda i: (0, 0)),
          pl.BlockSpec((blk, DIM), lambda i: (i, 0)),
          pl.BlockSpec((blk, 1), lambda i: (i, 0)),
          pl.BlockSpec((blk, 1), lambda i: (i, 0)),
          pl.BlockSpec((1, DIM), lambda i: (0, 0)),
          pl.BlockSpec((1, 1), lambda i: (0, 0)),
      ],
      out_specs=pl.BlockSpec((blk, DIM), lambda i: (i, 0)),
      out_shape=jax.ShapeDtypeStruct((NPAD, DIM), jnp.float32),
  )(num_part, den_part, h, a_s, a_d, bias.reshape(1, DIM),
    m_scalar.reshape(1, 1))


def kernel(x, edge_index, W, att_src, att_dst, bias):
  n_edges = edge_index.shape[1]
  per_tile = n_edges // NUM_TILES
  n_chunks = per_tile // CHUNK

  h, a_s, a_d, ms, md = _project(x, W, att_src, att_dst)

  m_raw = jnp.max(ms) + jnp.max(md)
  m_scalar = jnp.maximum(m_raw, 0.2 * m_raw)

  n_groups = 5
  grp_chunks = n_chunks // n_groups
  ei = edge_index.astype(jnp.int32)
  src4 = ei[0].reshape(NUM_TILES, n_groups, grp_chunks, CHUNK)
  dst4 = ei[1].reshape(NUM_TILES, n_groups, grp_chunks, CHUNK)

  m_arr = jnp.full((16,), m_scalar, jnp.float32)
  zrows = jnp.zeros((128, DIM), jnp.float32)
  z1d = jnp.zeros((NPAD,), jnp.float32)

  w4, den_part = _sc_weights(
      src4, dst4, a_s.reshape(NPAD), a_d.reshape(NPAD), m_arr, z1d)
  num_part = _sc_aggregate(src4, dst4, w4, h, zrows)

  out = _finalize(num_part, den_part, h, a_s, a_d, bias, m_scalar)
  return out[:N_NODES]


# speedup vs baseline: 1.0154x; 1.0154x over previous
import functools

import jax
import jax.numpy as jnp
from jax import lax
from jax.experimental import pallas as pl
from jax.experimental.pallas import tpu as pltpu
from jax.experimental.pallas import tpu_sc as plsc

N_NODES = 10000
DIM = 128

NUM_TILES = 32
CHUNK = 80
NPAD = 10240
ROWS_PER_TILE = NPAD // 16


def _mm_body(x_ref, w_ref, asrc_ref, adst_ref, h_ref, as_ref, ad_ref,
             ms_ref, md_ref):
  h = jax.lax.dot_general(x_ref[...], w_ref[...], (((1,), (0,)), ((), ())),
                          preferred_element_type=jnp.float32)
  h_ref[...] = h
  a_s = jnp.sum(h * asrc_ref[...], axis=1, keepdims=True)
  a_d = jnp.sum(h * adst_ref[...], axis=1, keepdims=True)
  as_ref[...] = a_s
  ad_ref[...] = a_d

  @pl.when(pl.program_id(0) == 0)
  def _():
    ms_ref[0, 0] = jnp.float32(-jnp.inf)
    md_ref[0, 0] = jnp.float32(-jnp.inf)

  ms_ref[0, 0] = jnp.maximum(ms_ref[0, 0], jnp.max(a_s))
  md_ref[0, 0] = jnp.maximum(md_ref[0, 0], jnp.max(a_d))


def _project(x, W, att_src, att_dst):
  blk = 1000
  grid = N_NODES // blk
  return pl.pallas_call(
      _mm_body,
      grid=(grid,),
      in_specs=[
          pl.BlockSpec((blk, DIM), lambda i: (i, 0)),
          pl.BlockSpec((DIM, DIM), lambda i: (0, 0)),
          pl.BlockSpec((1, DIM), lambda i: (0, 0)),
          pl.BlockSpec((1, DIM), lambda i: (0, 0)),
      ],
      out_specs=[
          pl.BlockSpec((blk, DIM), lambda i: (i, 0)),
          pl.BlockSpec((blk, 1), lambda i: (i, 0)),
          pl.BlockSpec((blk, 1), lambda i: (i, 0)),
          pl.BlockSpec((1, 1), lambda i: (0, 0), memory_space=pltpu.SMEM),
          pl.BlockSpec((1, 1), lambda i: (0, 0), memory_space=pltpu.SMEM),
      ],
      out_shape=[
          jax.ShapeDtypeStruct((NPAD, DIM), jnp.float32),
          jax.ShapeDtypeStruct((NPAD, 1), jnp.float32),
          jax.ShapeDtypeStruct((NPAD, 1), jnp.float32),
          jax.ShapeDtypeStruct((1, 1), jnp.float32),
          jax.ShapeDtypeStruct((1, 1), jnp.float32),
      ],
  )(x, W, att_src.reshape(1, DIM), att_dst.reshape(1, DIM))


def _sc_w_body(src_hbm, dst_hbm, asrc_hbm, adst_hbm, m_hbm, z1d_hbm,
               w_hbm, den_hbm,
               src_v, dst_v, w_v, asrc_v, adst_v, den_v, m_v):
  core = lax.axis_index("c")
  sub = lax.axis_index("s")
  wid = sub * 2 + core

  n_groups = src_hbm.shape[1]
  grp_chunks = src_hbm.shape[2]

  pltpu.sync_copy(asrc_hbm, asrc_v)
  pltpu.sync_copy(adst_hbm, adst_v)
  pltpu.sync_copy(z1d_hbm, den_v)
  pltpu.sync_copy(m_hbm, m_v)
  mv = m_v[...]

  def group_step(grp, carry):
    pltpu.sync_copy(src_hbm.at[wid, grp], src_v)
    pltpu.sync_copy(dst_hbm.at[wid, grp], dst_v)

    def chunk_step(j, carry2):
      for g in range(CHUNK // 16):
        s16 = src_v[j, pl.ds(g * 16, 16)]
        d16 = dst_v[j, pl.ds(g * 16, 16)]
        av = plsc.load_gather(asrc_v, [s16])
        bv = plsc.load_gather(adst_v, [d16])
        s = av + bv
        s = jnp.maximum(s, 0.2 * s)
        wv = jnp.exp(s - mv)
        plsc.addupdate_scatter(den_v, [d16], wv)
        w_v[j, pl.ds(g * 16, 16)] = wv
      return carry2

    lax.fori_loop(0, grp_chunks, chunk_step, 0)
    pltpu.sync_copy(w_v, w_hbm.at[wid, grp])
    return carry

  lax.fori_loop(0, n_groups, group_step, 0)
  pltpu.sync_copy(den_v, den_hbm.at[wid])


def _sc_weights(src4, dst4, a_src, a_dst, m_arr, z1d):
  grp_chunks = src4.shape[2]
  mesh = plsc.VectorSubcoreMesh(core_axis_name="c", subcore_axis_name="s")
  kern = functools.partial(
      pl.kernel,
      mesh=mesh,
      compiler_params=pltpu.CompilerParams(needs_layout_passes=False),
      out_type=[
          jax.ShapeDtypeStruct(src4.shape, jnp.float32),
          jax.ShapeDtypeStruct((NUM_TILES, NPAD), jnp.float32),
      ],
      scratch_types=[
          pltpu.VMEM((grp_chunks, CHUNK), jnp.int32),
          pltpu.VMEM((grp_chunks, CHUNK), jnp.int32),
          pltpu.VMEM((grp_chunks, CHUNK), jnp.float32),
          pltpu.VMEM((NPAD,), jnp.float32),
          pltpu.VMEM((NPAD,), jnp.float32),
          pltpu.VMEM((NPAD,), jnp.float32),
          pltpu.VMEM((16,), jnp.float32),
      ],
  )(_sc_w_body)
  return kern(src4, dst4, a_src, a_dst, m_arr, z1d)


NBUF = 3


def _sc_agg_body(src_hbm, dst_hbm, w_hbm, h_hbm, zrows_hbm,
                 num_hbm,
                 src_v, dst_v, w_v, rows0, rows1, rows2,
                 num_sh, sg0, sg1, sg2, ss0, ss1, ss2):
  core = lax.axis_index("c")
  sub = lax.axis_index("s")
  wid = sub * 2 + core

  n_groups = src_hbm.shape[1]
  grp_chunks = src_hbm.shape[2]
  rows = [rows0, rows1, rows2]
  sg = [sg0, sg1, sg2]
  ss = [ss0, ss1, ss2]

  row0 = sub * ROWS_PER_TILE
  for i in range(5):
    pltpu.sync_copy(zrows_hbm, num_sh.at[pl.ds(row0 + i * 128, 128)])
  plsc.subcore_barrier()

  def group_step(grp, carry):
    pltpu.sync_copy(src_hbm.at[wid, grp], src_v)
    pltpu.sync_copy(dst_hbm.at[wid, grp], dst_v)
    pltpu.sync_copy(w_hbm.at[wid, grp], w_v)

    cp_g = {}
    cp_s = {}
    ahead = NBUF - 2
    for j in range(grp_chunks):
      b = j % NBUF
      if j == 0:
        for k in range(min(ahead, grp_chunks)):
          cp_g[k] = pltpu.async_copy(
              h_hbm.at[src_v.at[k]], rows[k % NBUF], sg[k % NBUF])
      cp_g[j].wait()
      if j + ahead < grp_chunks:
        b2 = (j + ahead) % NBUF
        if j + ahead >= NBUF:
          cp_s[j + ahead - NBUF].wait()
        cp_g[j + ahead] = pltpu.async_copy(
            h_hbm.at[src_v.at[j + ahead]], rows[b2], sg[b2])

      jv = jnp.full((16,), j, jnp.int32)

      def scale_row(r, c, b=b, jv=jv):
        wb = plsc.load_gather(w_v, [jv, jnp.full((16,), r, jnp.int32)])
        for d in range(DIM // 16):
          rows[b][r, pl.ds(d * 16, 16)] = rows[b][r, pl.ds(d * 16, 16)] * wb
        return c

      lax.fori_loop(0, CHUNK, scale_row, 0, unroll=8)

      cp_s[j] = pltpu.async_copy(rows[b], num_sh.at[dst_v.at[j]], ss[b],
                                 add=True)

    for j in range(max(0, grp_chunks - NBUF), grp_chunks):
      cp_s[j].wait()
    return carry

  lax.fori_loop(0, n_groups, group_step, 0)

  plsc.subcore_barrier()
  pltpu.sync_copy(num_sh.at[pl.ds(row0, ROWS_PER_TILE)],
                  num_hbm.at[core, pl.ds(row0, ROWS_PER_TILE)])


def _sc_aggregate(src4, dst4, w4, h, zrows):
  grp_chunks = src4.shape[2]
  mesh = plsc.VectorSubcoreMesh(core_axis_name="c", subcore_axis_name="s")
  kern = functools.partial(
      pl.kernel,
      mesh=mesh,
      compiler_params=pltpu.CompilerParams(needs_layout_passes=False),
      out_type=jax.ShapeDtypeStruct((2, NPAD, DIM), jnp.float32),
      scratch_types=[
          pltpu.VMEM((grp_chunks, CHUNK), jnp.int32),
          pltpu.VMEM((grp_chunks, CHUNK), jnp.int32),
          pltpu.VMEM((grp_chunks, CHUNK), jnp.float32),
          pltpu.VMEM((CHUNK, DIM), jnp.float32),
          pltpu.VMEM((CHUNK, DIM), jnp.float32),
          pltpu.VMEM((CHUNK, DIM), jnp.float32),
          pltpu.VMEM_SHARED((NPAD, DIM), jnp.float32),
          pltpu.SemaphoreType.DMA,
          pltpu.SemaphoreType.DMA,
          pltpu.SemaphoreType.DMA,
          pltpu.SemaphoreType.DMA,
          pltpu.SemaphoreType.DMA,
          pltpu.SemaphoreType.DMA,
      ],
  )(_sc_agg_body)
  return kern(src4, dst4, w4, h, zrows)


def _fin_body(num_ref, den_ref, h_ref, as_ref, ad_ref, bias_ref, m_ref,
              out_ref):
  s = as_ref[...] + ad_ref[...]
  s = jnp.maximum(s, 0.2 * s)
  ws = jnp.exp(s - m_ref[0, 0])
  num = num_ref[0] + num_ref[1] + ws * h_ref[...]
  blk = h_ref.shape[0]
  den_blk = den_ref[:, pl.ds(pl.program_id(0) * blk, blk)]
  den = jnp.sum(den_blk, axis=0)[:, None] + ws + 1e-16
  out_ref[...] = num / den + bias_ref[...]


def _finalize(num_part, den_part, h, a_s, a_d, bias, m_scalar):
  blk = 1024
  grid = NPAD // blk
  return pl.pallas_call(
      _fin_body,
      grid=(grid,),
      in_specs=[
          pl.BlockSpec((2, blk, DIM), lambda i: (0, i, 0)),
          pl.BlockSpec((NUM_TILES, NPAD), lambda i: (0, 0)),
          pl.BlockSpec((blk, DIM), lambda i: (i, 0)),
          pl.BlockSpec((blk, 1), lambda i: (i, 0)),
          pl.BlockSpec((blk, 1), lambda i: (i, 0)),
          pl.BlockSpec((1, DIM), lambda i: (0, 0)),
          pl.BlockSpec((1, 1), lambda i: (0, 0)),
      ],
      out_specs=pl.BlockSpec((blk, DIM), lambda i: (i, 0)),
      out_shape=jax.ShapeDtypeStruct((NPAD, DIM), jnp.float32),
  )(num_part, den_part, h, a_s, a_d, bias.reshape(1, DIM),
    m_scalar.reshape(1, 1))


def kernel(x, edge_index, W, att_src, att_dst, bias):
  n_edges = edge_index.shape[1]
  per_tile = n_edges // NUM_TILES
  n_chunks = per_tile // CHUNK

  h, a_s, a_d, ms, md = _project(x, W, att_src, att_dst)

  m_raw = jnp.max(ms) + jnp.max(md)
  m_scalar = jnp.maximum(m_raw, 0.2 * m_raw)

  n_groups = 5
  grp_chunks = n_chunks // n_groups
  ei = edge_index.astype(jnp.int32)
  src4 = ei[0].reshape(NUM_TILES, n_groups, grp_chunks, CHUNK)
  dst4 = ei[1].reshape(NUM_TILES, n_groups, grp_chunks, CHUNK)

  m_arr = jnp.full((16,), m_scalar, jnp.float32)
  zrows = jnp.zeros((128, DIM), jnp.float32)
  z1d = jnp.zeros((NPAD,), jnp.float32)

  w4, den_part = _sc_weights(
      src4, dst4, a_s.reshape(NPAD), a_d.reshape(NPAD), m_arr, z1d)
  num_part = _sc_aggregate(src4, dst4, w4, h, zrows)

  out = _finalize(num_part, den_part, h, a_s, a_d, bias, m_scalar)
  return out[:N_NODES]


# speedup vs baseline: 1.0408x; 1.0250x over previous
import functools

import jax
import jax.numpy as jnp
from jax import lax
from jax.experimental import pallas as pl
from jax.experimental.pallas import tpu as pltpu
from jax.experimental.pallas import tpu_sc as plsc

N_NODES = 10000
DIM = 128

NUM_TILES = 32
CHUNK = 80
NPAD = 10240
ROWS_PER_TILE = NPAD // 16


def _mm_body(x_ref, w_ref, asrc_ref, adst_ref, h_ref, as_ref, ad_ref,
             ms_ref, md_ref):
  h = jax.lax.dot_general(x_ref[...], w_ref[...], (((1,), (0,)), ((), ())),
                          preferred_element_type=jnp.float32)
  h_ref[...] = h
  a_s = jnp.sum(h * asrc_ref[...], axis=1, keepdims=True)
  a_d = jnp.sum(h * adst_ref[...], axis=1, keepdims=True)
  as_ref[...] = a_s
  ad_ref[...] = a_d

  @pl.when(pl.program_id(0) == 0)
  def _():
    ms_ref[0, 0] = jnp.float32(-jnp.inf)
    md_ref[0, 0] = jnp.float32(-jnp.inf)

  ms_ref[0, 0] = jnp.maximum(ms_ref[0, 0], jnp.max(a_s))
  md_ref[0, 0] = jnp.maximum(md_ref[0, 0], jnp.max(a_d))


def _project(x, W, att_src, att_dst):
  blk = 1000
  grid = N_NODES // blk
  return pl.pallas_call(
      _mm_body,
      grid=(grid,),
      in_specs=[
          pl.BlockSpec((blk, DIM), lambda i: (i, 0)),
          pl.BlockSpec((DIM, DIM), lambda i: (0, 0)),
          pl.BlockSpec((1, DIM), lambda i: (0, 0)),
          pl.BlockSpec((1, DIM), lambda i: (0, 0)),
      ],
      out_specs=[
          pl.BlockSpec((blk, DIM), lambda i: (i, 0)),
          pl.BlockSpec((blk, 1), lambda i: (i, 0)),
          pl.BlockSpec((blk, 1), lambda i: (i, 0)),
          pl.BlockSpec((1, 1), lambda i: (0, 0), memory_space=pltpu.SMEM),
          pl.BlockSpec((1, 1), lambda i: (0, 0), memory_space=pltpu.SMEM),
      ],
      out_shape=[
          jax.ShapeDtypeStruct((NPAD, DIM), jnp.float32),
          jax.ShapeDtypeStruct((NPAD, 1), jnp.float32),
          jax.ShapeDtypeStruct((NPAD, 1), jnp.float32),
          jax.ShapeDtypeStruct((1, 1), jnp.float32),
          jax.ShapeDtypeStruct((1, 1), jnp.float32),
      ],
  )(x, W, att_src.reshape(1, DIM), att_dst.reshape(1, DIM))


def _sc_w_body(src_hbm, dst_hbm, asrc_hbm, adst_hbm, m_hbm, z1d_hbm,
               w_hbm, den_hbm,
               src_v0, dst_v0, src_v1, dst_v1, w_v, asrc_v, adst_v, den_v,
               m_v, sw0, sw1):
  core = lax.axis_index("c")
  sub = lax.axis_index("s")
  wid = sub * 2 + core

  n_groups = src_hbm.shape[1]
  grp_chunks = src_hbm.shape[2]
  swin = [src_v0, src_v1]
  dwin = [dst_v0, dst_v1]
  sems = [sw0, sw1]

  pltpu.sync_copy(asrc_hbm, asrc_v)
  pltpu.sync_copy(adst_hbm, adst_v)
  pltpu.sync_copy(z1d_hbm, den_v)
  pltpu.sync_copy(m_hbm, m_v)
  mv = m_v[...]

  cp_w = {}
  cp_w[0] = (pltpu.async_copy(src_hbm.at[wid, 0], swin[0], sems[0]),
             pltpu.async_copy(dst_hbm.at[wid, 0], dwin[0], sems[0]))

  for grp in range(n_groups):
    p = grp % 2
    cp_w[grp][0].wait()
    cp_w[grp][1].wait()
    if grp + 1 < n_groups:
      q = 1 - p
      cp_w[grp + 1] = (
          pltpu.async_copy(src_hbm.at[wid, grp + 1], swin[q], sems[q]),
          pltpu.async_copy(dst_hbm.at[wid, grp + 1], dwin[q], sems[q]))
    src_v = swin[p]
    dst_v = dwin[p]

    def chunk_step(j, carry2, src_v=src_v, dst_v=dst_v):
      for g in range(CHUNK // 16):
        s16 = src_v[j, pl.ds(g * 16, 16)]
        d16 = dst_v[j, pl.ds(g * 16, 16)]
        av = plsc.load_gather(asrc_v, [s16])
        bv = plsc.load_gather(adst_v, [d16])
        s = av + bv
        s = jnp.maximum(s, 0.2 * s)
        wv = jnp.exp(s - mv)
        plsc.addupdate_scatter(den_v, [d16], wv)
        w_v[j, pl.ds(g * 16, 16)] = wv
      return carry2

    lax.fori_loop(0, grp_chunks, chunk_step, 0)
    pltpu.sync_copy(w_v, w_hbm.at[wid, grp])

  pltpu.sync_copy(den_v, den_hbm.at[wid])


def _sc_weights(src4, dst4, a_src, a_dst, m_arr, z1d):
  grp_chunks = src4.shape[2]
  mesh = plsc.VectorSubcoreMesh(core_axis_name="c", subcore_axis_name="s")
  kern = functools.partial(
      pl.kernel,
      mesh=mesh,
      compiler_params=pltpu.CompilerParams(needs_layout_passes=False),
      out_type=[
          jax.ShapeDtypeStruct(src4.shape, jnp.float32),
          jax.ShapeDtypeStruct((NUM_TILES, NPAD), jnp.float32),
      ],
      scratch_types=[
          pltpu.VMEM((grp_chunks, CHUNK), jnp.int32),
          pltpu.VMEM((grp_chunks, CHUNK), jnp.int32),
          pltpu.VMEM((grp_chunks, CHUNK), jnp.int32),
          pltpu.VMEM((grp_chunks, CHUNK), jnp.int32),
          pltpu.VMEM((grp_chunks, CHUNK), jnp.float32),
          pltpu.VMEM((NPAD,), jnp.float32),
          pltpu.VMEM((NPAD,), jnp.float32),
          pltpu.VMEM((NPAD,), jnp.float32),
          pltpu.VMEM((16,), jnp.float32),
          pltpu.SemaphoreType.DMA,
          pltpu.SemaphoreType.DMA,
      ],
  )(_sc_w_body)
  return kern(src4, dst4, a_src, a_dst, m_arr, z1d)


NBUF = 3


def _sc_agg_body(src_hbm, dst_hbm, w_hbm, h_hbm, zrows_hbm,
                 num_hbm,
                 src_v, dst_v, w_v, rows0, rows1, rows2,
                 num_sh, sg0, sg1, sg2, ss0, ss1, ss2):
  core = lax.axis_index("c")
  sub = lax.axis_index("s")
  wid = sub * 2 + core

  n_groups = src_hbm.shape[1]
  grp_chunks = src_hbm.shape[2]
  rows = [rows0, rows1, rows2]
  sg = [sg0, sg1, sg2]
  ss = [ss0, ss1, ss2]

  row0 = sub * ROWS_PER_TILE
  for i in range(5):
    pltpu.sync_copy(zrows_hbm, num_sh.at[pl.ds(row0 + i * 128, 128)])
  plsc.subcore_barrier()

  def group_step(grp, carry):
    pltpu.sync_copy(src_hbm.at[wid, grp], src_v)
    pltpu.sync_copy(dst_hbm.at[wid, grp], dst_v)
    pltpu.sync_copy(w_hbm.at[wid, grp], w_v)

    cp_g = {}
    cp_s = {}
    ahead = NBUF - 2
    for j in range(grp_chunks):
      b = j % NBUF
      if j == 0:
        for k in range(min(ahead, grp_chunks)):
          cp_g[k] = pltpu.async_copy(
              h_hbm.at[src_v.at[k]], rows[k % NBUF], sg[k % NBUF])
      cp_g[j].wait()
      if j + ahead < grp_chunks:
        b2 = (j + ahead) % NBUF
        if j + ahead >= NBUF:
          cp_s[j + ahead - NBUF].wait()
        cp_g[j + ahead] = pltpu.async_copy(
            h_hbm.at[src_v.at[j + ahead]], rows[b2], sg[b2])

      jv = jnp.full((16,), j, jnp.int32)

      def scale_row(r, c, b=b, jv=jv):
        wb = plsc.load_gather(w_v, [jv, jnp.full((16,), r, jnp.int32)])
        for d in range(DIM // 16):
          rows[b][r, pl.ds(d * 16, 16)] = rows[b][r, pl.ds(d * 16, 16)] * wb
        return c

      lax.fori_loop(0, CHUNK, scale_row, 0, unroll=8)

      cp_s[j] = pltpu.async_copy(rows[b], num_sh.at[dst_v.at[j]], ss[b],
                                 add=True)

    for j in range(max(0, grp_chunks - NBUF), grp_chunks):
      cp_s[j].wait()
    return carry

  lax.fori_loop(0, n_groups, group_step, 0)

  plsc.subcore_barrier()
  pltpu.sync_copy(num_sh.at[pl.ds(row0, ROWS_PER_TILE)],
                  num_hbm.at[core, pl.ds(row0, ROWS_PER_TILE)])


def _sc_aggregate(src4, dst4, w4, h, zrows):
  grp_chunks = src4.shape[2]
  mesh = plsc.VectorSubcoreMesh(core_axis_name="c", subcore_axis_name="s")
  kern = functools.partial(
      pl.kernel,
      mesh=mesh,
      compiler_params=pltpu.CompilerParams(needs_layout_passes=False),
      out_type=jax.ShapeDtypeStruct((2, NPAD, DIM), jnp.float32),
      scratch_types=[
          pltpu.VMEM((grp_chunks, CHUNK), jnp.int32),
          pltpu.VMEM((grp_chunks, CHUNK), jnp.int32),
          pltpu.VMEM((grp_chunks, CHUNK), jnp.float32),
          pltpu.VMEM((CHUNK, DIM), jnp.float32),
          pltpu.VMEM((CHUNK, DIM), jnp.float32),
          pltpu.VMEM((CHUNK, DIM), jnp.float32),
          pltpu.VMEM_SHARED((NPAD, DIM), jnp.float32),
          pltpu.SemaphoreType.DMA,
          pltpu.SemaphoreType.DMA,
          pltpu.SemaphoreType.DMA,
          pltpu.SemaphoreType.DMA,
          pltpu.SemaphoreType.DMA,
          pltpu.SemaphoreType.DMA,
      ],
  )(_sc_agg_body)
  return kern(src4, dst4, w4, h, zrows)


def _fin_body(num_ref, den_ref, h_ref, as_ref, ad_ref, bias_ref, m_ref,
              out_ref):
  s = as_ref[...] + ad_ref[...]
  s = jnp.maximum(s, 0.2 * s)
  ws = jnp.exp(s - m_ref[0, 0])
  num = num_ref[0] + num_ref[1] + ws * h_ref[...]
  blk = h_ref.shape[0]
  den_blk = den_ref[:, pl.ds(pl.program_id(0) * blk, blk)]
  den = jnp.sum(den_blk, axis=0)[:, None] + ws + 1e-16
  out_ref[...] = num / den + bias_ref[...]


def _finalize(num_part, den_part, h, a_s, a_d, bias, m_scalar):
  blk = 1024
  grid = NPAD // blk
  return pl.pallas_call(
      _fin_body,
      grid=(grid,),
      in_specs=[
          pl.BlockSpec((2, blk, DIM), lambda i: (0, i, 0)),
          pl.BlockSpec((NUM_TILES, NPAD), lambda i: (0, 0)),
          pl.BlockSpec((blk, DIM), lambda i: (i, 0)),
          pl.BlockSpec((blk, 1), lambda i: (i, 0)),
          pl.BlockSpec((blk, 1), lambda i: (i, 0)),
          pl.BlockSpec((1, DIM), lambda i: (0, 0)),
          pl.BlockSpec((1, 1), lambda i: (0, 0)),
      ],
      out_specs=pl.BlockSpec((blk, DIM), lambda i: (i, 0)),
      out_shape=jax.ShapeDtypeStruct((NPAD, DIM), jnp.float32),
  )(num_part, den_part, h, a_s, a_d, bias.reshape(1, DIM),
    m_scalar.reshape(1, 1))


def kernel(x, edge_index, W, att_src, att_dst, bias):
  n_edges = edge_index.shape[1]
  per_tile = n_edges // NUM_TILES
  n_chunks = per_tile // CHUNK

  h, a_s, a_d, ms, md = _project(x, W, att_src, att_dst)

  m_raw = jnp.max(ms) + jnp.max(md)
  m_scalar = jnp.maximum(m_raw, 0.2 * m_raw)

  n_groups = 5
  grp_chunks = n_chunks // n_groups
  ei = edge_index.astype(jnp.int32)
  src4 = ei[0].reshape(NUM_TILES, n_groups, grp_chunks, CHUNK)
  dst4 = ei[1].reshape(NUM_TILES, n_groups, grp_chunks, CHUNK)

  m_arr = jnp.full((16,), m_scalar, jnp.float32)
  zrows = jnp.zeros((128, DIM), jnp.float32)
  z1d = jnp.zeros((NPAD,), jnp.float32)

  w4, den_part = _sc_weights(
      src4, dst4, a_s.reshape(NPAD), a_d.reshape(NPAD), m_arr, z1d)
  num_part = _sc_aggregate(src4, dst4, w4, h, zrows)

  out = _finalize(num_part, den_part, h, a_s, a_d, bias, m_scalar)
  return out[:N_NODES]
